# baseline (speedup 1.0000x reference)
import functools

import jax
import jax.numpy as jnp
import numpy as np
from jax.experimental import pallas as pl
from jax.experimental.pallas import tpu as pltpu

_H0, _W0 = 48, 48
_HT, _WT = 96, 96


def _resize_mat() -> np.ndarray:
    R = np.zeros((_HT, _H0), dtype=np.float64)
    for o in range(_HT):
        s = 0.5 * o - 0.25
        sc = min(max(s, 0.0), _H0 - 1.0)
        i0 = min(int(np.floor(sc)), _H0 - 2)
        w1 = sc - i0
        R[o, i0] += 1.0 - w1
        R[o, i0 + 1] += w1
    return R.astype(np.float32)


def _taps(p):
    s = 0.5 * p.astype(jnp.float32) - 0.25
    sc = jnp.clip(s, 0.0, _H0 - 1.0)
    i0 = jnp.minimum(jnp.floor(sc).astype(jnp.int32), _H0 - 2)
    w1 = sc - i0.astype(jnp.float32)
    return i0, 1.0 - w1, w1


def _w2d(py, px):
    iy0, wy0, wy1 = _taps(py)
    ix0, wx0, wx1 = _taps(px)
    ri = jax.lax.broadcasted_iota(jnp.int32, (_H0, _W0), 0)
    ci = jax.lax.broadcasted_iota(jnp.int32, (_H0, _W0), 1)
    wy = jnp.where(ri == iy0, wy0, 0.0) + jnp.where(ri == iy0 + 1, wy1, 0.0)
    wx = jnp.where(ci == ix0, wx0, 0.0) + jnp.where(ci == ix0 + 1, wx1, 0.0)
    return wy * wx


def _decode_kernel(hm_ref, ct_ref, rg_ref, of_ref, r_ref, c_ref, out_ref, *, nj):
    f32 = jnp.float32
    R = r_ref[...]
    C = c_ref[...]
    hp = jax.lax.Precision.HIGHEST

    hm = hm_ref[0]
    ct = ct_ref[0, 0]
    rg = rg_ref[0]
    of = of_ref[0]

    dot = functools.partial(jnp.dot, precision=hp, preferred_element_type=f32)

    ct_r = dot(R, dot(ct, C))
    hm1 = dot(hm.reshape(nj * _H0, _W0), C)
    hm_r = [dot(R, hm1[n * _H0:(n + 1) * _H0]) for n in range(nj)]

    li = (jax.lax.broadcasted_iota(jnp.int32, (_HT, _WT), 0) * _WT
          + jax.lax.broadcasted_iota(jnp.int32, (_HT, _WT), 1))
    xi = jax.lax.broadcasted_iota(jnp.int32, (_HT, _WT), 1).astype(f32)
    yi = jax.lax.broadcasted_iota(jnp.int32, (_HT, _WT), 0).astype(f32)

    m = jnp.max(ct_r)
    idx = jnp.min(jnp.where(ct_r == m, li, _HT * _WT))
    cy = idx // _WT
    cx = idx - cy * _WT

    wc = _w2d(cy, cx)
    cx_f = cx.astype(f32)
    cy_f = cy.astype(f32)

    vals = []
    for n in range(nj):
        rx = jnp.sum(rg[2 * n] * wc)
        ry = jnp.sum(rg[2 * n + 1] * wc)
        reg_x = jnp.clip(cx_f + rx + 0.5, 0.0, _WT - 1.0)
        reg_y = jnp.clip(cy_f + ry + 0.5, 0.0, _HT - 1.0)
        d2 = (xi - reg_x) ** 2 + (yi - reg_y) ** 2 + 1e-9
        h = hm_r[n]
        q = h * h / d2
        m2 = jnp.max(q)
        idx2 = jnp.min(jnp.where(q == m2, li, _HT * _WT))
        jy = idx2 // _WT
        jx = idx2 - jy * _WT
        score = jnp.sum(jnp.where(li == idx2, h, 0.0))
        wj = _w2d(jy, jx)
        ox = jnp.sum(of[2 * n] * wj)
        oy = jnp.sum(of[2 * n + 1] * wj)
        vals.append((jx.astype(f32) + ox) / float(_WT))
        vals.append((jy.astype(f32) + oy) / float(_HT))
        vals.append(score)

    lane = jax.lax.broadcasted_iota(jnp.int32, (1, 1, 3 * nj), 2)
    acc = jnp.zeros((1, 1, 3 * nj), f32)
    for k, v in enumerate(vals):
        acc = acc + jnp.where(lane == k, v, 0.0)
    out_ref[...] = acc


def kernel(hm, ct, rg, of):
    B, nj = hm.shape[0], hm.shape[1]
    Rm = jnp.asarray(_resize_mat())
    Cm = jnp.asarray(_resize_mat().T)
    out = pl.pallas_call(
        functools.partial(_decode_kernel, nj=nj),
        grid=(B,),
        in_specs=[
            pl.BlockSpec((1, nj, _H0, _W0), lambda b: (b, 0, 0, 0)),
            pl.BlockSpec((1, 1, _H0, _W0), lambda b: (b, 0, 0, 0)),
            pl.BlockSpec((1, 2 * nj, _H0, _W0), lambda b: (b, 0, 0, 0)),
            pl.BlockSpec((1, 2 * nj, _H0, _W0), lambda b: (b, 0, 0, 0)),
            pl.BlockSpec((_HT, _H0), lambda b: (0, 0)),
            pl.BlockSpec((_H0, _WT), lambda b: (0, 0)),
        ],
        out_specs=pl.BlockSpec((1, 1, 3 * nj), lambda b: (b, 0, 0)),
        out_shape=jax.ShapeDtypeStruct((B, 1, 3 * nj), jnp.float32),
        compiler_params=pltpu.CompilerParams(
            dimension_semantics=("arbitrary",),
        ),
    )(hm, ct, rg, of, Rm, Cm)
    return out.reshape(B, 3 * nj)


# speedup vs baseline: 2.2619x; 2.2619x over previous
import functools

import jax
import jax.numpy as jnp
import numpy as np
from jax.experimental import pallas as pl
from jax.experimental.pallas import tpu as pltpu

_H0, _W0 = 48, 48
_HT, _WT = 96, 96
_BB = 4


def _wresize_mat() -> np.ndarray:
    C = np.zeros((_W0, _WT), dtype=np.float64)
    for o in range(_WT):
        s = 0.5 * o - 0.25
        sc = min(max(s, 0.0), _W0 - 1.0)
        i0 = min(int(np.floor(sc)), _W0 - 2)
        w1 = sc - i0
        C[i0, o] += 1.0 - w1
        C[i0 + 1, o] += w1
    return C.astype(np.float32)


def _taps(p):
    s = 0.5 * p.astype(jnp.float32) - 0.25
    sc = jnp.clip(s, 0.0, _H0 - 1.0)
    i0 = jnp.minimum(jnp.floor(sc).astype(jnp.int32), _H0 - 2)
    w1 = sc - i0.astype(jnp.float32)
    return i0, 1.0 - w1, w1


def _w2d(py, px, shape):
    nlead = len(shape) - 2
    exp = (Ellipsis,) + (None,) * 2
    iy0, wy0, wy1 = _taps(py)
    ix0, wx0, wx1 = _taps(px)
    iy0, wy0, wy1 = iy0[exp], wy0[exp], wy1[exp]
    ix0, wx0, wx1 = ix0[exp], wx0[exp], wx1[exp]
    ri = jax.lax.broadcasted_iota(jnp.int32, shape, nlead)
    ci = jax.lax.broadcasted_iota(jnp.int32, shape, nlead + 1)
    wy = jnp.where(ri == iy0, wy0, 0.0) + jnp.where(ri == iy0 + 1, wy1, 0.0)
    wx = jnp.where(ci == ix0, wx0, 0.0) + jnp.where(ci == ix0 + 1, wx1, 0.0)
    return wy * wx


def _hresize(a):
    up = jnp.concatenate([a[..., :1, :], a[..., :-1, :]], axis=-2)
    dn = jnp.concatenate([a[..., 1:, :], a[..., -1:, :]], axis=-2)
    even = 0.75 * a + 0.25 * up
    odd = 0.75 * a + 0.25 * dn
    inter = jnp.stack([even, odd], axis=-2)
    return inter.reshape(a.shape[:-2] + (2 * a.shape[-2], a.shape[-1]))


def _decode_kernel(hm_ref, ct_ref, rg_ref, of_ref, c_ref, x_ref, y_ref, s_ref,
                   *, nj, bb):
    f32 = jnp.float32
    C = c_ref[...]
    nc = nj + 1

    hm = hm_ref[...]
    ct = ct_ref[...]
    rg = rg_ref[...]
    of = of_ref[...]

    x_in = jnp.concatenate([ct, hm], axis=1)
    a = jnp.dot(x_in.reshape(bb * nc * _H0, _W0), C,
                precision=jax.lax.Precision.HIGHEST,
                preferred_element_type=f32)
    f = _hresize(a.reshape(bb, nc, _H0, _WT))
    ct_r = f[:, 0]
    hm_r = f[:, 1:]

    li = (jax.lax.broadcasted_iota(jnp.int32, (_HT, _WT), 0) * _WT
          + jax.lax.broadcasted_iota(jnp.int32, (_HT, _WT), 1))
    big = _HT * _WT

    m = jnp.max(ct_r, axis=(1, 2))
    idx = jnp.min(jnp.where(ct_r == m[:, None, None], li[None], big),
                  axis=(1, 2))
    cy = idx // _WT
    cx = idx - cy * _WT

    wc = _w2d(cy, cx, (bb, _H0, _W0))
    rxy = jnp.sum(rg * wc[:, None], axis=(2, 3))
    rxy = rxy.reshape(bb, nj, 2)
    reg_x = jnp.clip(cx.astype(f32)[:, None] + rxy[:, :, 0] + 0.5,
                     0.0, _WT - 1.0)
    reg_y = jnp.clip(cy.astype(f32)[:, None] + rxy[:, :, 1] + 0.5,
                     0.0, _HT - 1.0)

    xi = jax.lax.broadcasted_iota(jnp.int32, (_HT, _WT), 1).astype(f32)
    yi = jax.lax.broadcasted_iota(jnp.int32, (_HT, _WT), 0).astype(f32)
    d2 = ((xi[None, None] - reg_x[:, :, None, None]) ** 2
          + (yi[None, None] - reg_y[:, :, None, None]) ** 2 + 1e-9)
    t = hm_r * jax.lax.rsqrt(d2)
    m2 = jnp.max(t, axis=(2, 3))
    idx2 = jnp.min(jnp.where(t == m2[:, :, None, None], li[None, None], big),
                   axis=(2, 3))
    jy = idx2 // _WT
    jx = idx2 - jy * _WT
    score = jnp.sum(jnp.where(li[None, None] == idx2[:, :, None, None],
                              hm_r, 0.0), axis=(2, 3))

    wj = _w2d(jy, jx, (bb, nj, _H0, _W0))
    oxy = jnp.sum(of.reshape(bb, nj, 2, _H0, _W0) * wj[:, :, None],
                  axis=(3, 4))

    x_ref[:, 0, :] = (jx.astype(f32) + oxy[:, :, 0]) / float(_WT)
    y_ref[:, 0, :] = (jy.astype(f32) + oxy[:, :, 1]) / float(_HT)
    s_ref[:, 0, :] = score


def kernel(hm, ct, rg, of):
    B, nj = hm.shape[0], hm.shape[1]
    bb = _BB
    Cm = jnp.asarray(_wresize_mat())
    spec3 = lambda c: pl.BlockSpec((bb, c, _H0, _W0), lambda b: (b, 0, 0, 0))
    x, y, s = pl.pallas_call(
        functools.partial(_decode_kernel, nj=nj, bb=bb),
        grid=(B // bb,),
        in_specs=[
            spec3(nj),
            spec3(1),
            spec3(2 * nj),
            spec3(2 * nj),
            pl.BlockSpec((_W0, _WT), lambda b: (0, 0)),
        ],
        out_specs=[
            pl.BlockSpec((bb, 1, nj), lambda b: (b, 0, 0)),
            pl.BlockSpec((bb, 1, nj), lambda b: (b, 0, 0)),
            pl.BlockSpec((bb, 1, nj), lambda b: (b, 0, 0)),
        ],
        out_shape=[jax.ShapeDtypeStruct((B, 1, nj), jnp.float32)] * 3,
        compiler_params=pltpu.CompilerParams(
            dimension_semantics=("arbitrary",),
        ),
    )(hm, ct, rg, of, Cm)
    return jnp.stack([x[:, 0], y[:, 0], s[:, 0]], axis=2).reshape(B, 3 * nj)


# speedup vs baseline: 3.3089x; 1.4629x over previous
import functools

import jax
import jax.numpy as jnp
import numpy as np
from jax.experimental import pallas as pl
from jax.experimental.pallas import tpu as pltpu

_H0, _W0 = 48, 48
_HT, _WT = 96, 96
_BB = 4


def _wresize_mat() -> np.ndarray:
    C = np.zeros((_W0, _WT), dtype=np.float64)
    for o in range(_WT):
        s = 0.5 * o - 0.25
        sc = min(max(s, 0.0), _W0 - 1.0)
        i0 = min(int(np.floor(sc)), _W0 - 2)
        w1 = sc - i0
        C[i0, o] += 1.0 - w1
        C[i0 + 1, o] += w1
    return C.astype(np.float32)


def _taps(p):
    s = 0.5 * p.astype(jnp.float32) - 0.25
    sc = jnp.clip(s, 0.0, _H0 - 1.0)
    i0 = jnp.minimum(jnp.floor(sc).astype(jnp.int32), _H0 - 2)
    w1 = sc - i0.astype(jnp.float32)
    return i0, 1.0 - w1, w1


def _w2d(py, px, shape):
    nlead = len(shape) - 2
    exp = (Ellipsis,) + (None,) * 2
    iy0, wy0, wy1 = _taps(py)
    ix0, wx0, wx1 = _taps(px)
    iy0, wy0, wy1 = iy0[exp], wy0[exp], wy1[exp]
    ix0, wx0, wx1 = ix0[exp], wx0[exp], wx1[exp]
    ri = jax.lax.broadcasted_iota(jnp.int32, shape, nlead)
    ci = jax.lax.broadcasted_iota(jnp.int32, shape, nlead + 1)
    wy = jnp.where(ri == iy0, wy0, 0.0) + jnp.where(ri == iy0 + 1, wy1, 0.0)
    wx = jnp.where(ci == ix0, wx0, 0.0) + jnp.where(ci == ix0 + 1, wx1, 0.0)
    return wy * wx


def _hresize_parity(a):
    up = jnp.concatenate([a[..., :1, :], a[..., :-1, :]], axis=-2)
    dn = jnp.concatenate([a[..., 1:, :], a[..., -1:, :]], axis=-2)
    even = 0.75 * a + 0.25 * up
    odd = 0.75 * a + 0.25 * dn
    return jnp.stack([even, odd], axis=-3)


def _decode_kernel(hm_ref, ct_ref, rg_ref, of_ref, c_ref, x_ref, y_ref, s_ref,
                   *, nj, bb):
    f32 = jnp.float32
    C = c_ref[...]
    nc = nj + 1

    hm = hm_ref[...]
    ct = ct_ref[...]
    rg = rg_ref[...]
    of = of_ref[...]

    x_in = jnp.concatenate([ct, hm], axis=1)
    a = jnp.dot(x_in.reshape(bb * nc * _H0, _W0), C,
                precision=jax.lax.Precision.HIGHEST,
                preferred_element_type=f32)
    f = _hresize_parity(a.reshape(bb, nc, _H0, _WT))
    ct_r = f[:, 0]
    hm_r = f[:, 1:]

    pshape = (2, _H0, _WT)
    yrow = (jax.lax.broadcasted_iota(jnp.int32, pshape, 1) * 2
            + jax.lax.broadcasted_iota(jnp.int32, pshape, 0))
    li = yrow * _WT + jax.lax.broadcasted_iota(jnp.int32, pshape, 2)
    big = _HT * _WT

    m = jnp.max(ct_r, axis=(1, 2, 3))
    idx = jnp.min(jnp.where(ct_r == m[:, None, None, None], li[None], big),
                  axis=(1, 2, 3))
    cy = idx // _WT
    cx = idx - cy * _WT

    wc = _w2d(cy, cx, (bb, _H0, _W0))
    rxy = jnp.sum(rg * wc[:, None], axis=(2, 3))
    rxy = rxy.reshape(bb, nj, 2)
    reg_x = jnp.clip(cx.astype(f32)[:, None] + rxy[:, :, 0] + 0.5,
                     0.0, _WT - 1.0)
    reg_y = jnp.clip(cy.astype(f32)[:, None] + rxy[:, :, 1] + 0.5,
                     0.0, _HT - 1.0)

    yr = (jax.lax.broadcasted_iota(jnp.int32, (2, _H0, 1), 1) * 2
          + jax.lax.broadcasted_iota(jnp.int32, (2, _H0, 1), 0)).astype(f32)
    xr = jax.lax.broadcasted_iota(jnp.int32, (1, 1, _WT), 2).astype(f32)
    dy2 = (yr[None, None] - reg_y[:, :, None, None, None]) ** 2 + 1e-9
    dx2 = (xr[None, None] - reg_x[:, :, None, None, None]) ** 2
    t = hm_r * jax.lax.rsqrt(dy2 + dx2)
    m2 = jnp.max(t, axis=(2, 3, 4))
    idx2 = jnp.min(jnp.where(t == m2[:, :, None, None, None],
                             li[None, None], big), axis=(2, 3, 4))
    jy = idx2 // _WT
    jx = idx2 - jy * _WT
    jyf = jy.astype(f32)
    jxf = jx.astype(f32)
    d2p = (jyf - reg_y) ** 2 + 1e-9 + (jxf - reg_x) ** 2
    score = m2 * jnp.sqrt(d2p)

    wj = _w2d(jy, jx, (bb, nj, _H0, _W0))
    oxy = jnp.sum(of.reshape(bb, nj, 2, _H0, _W0) * wj[:, :, None],
                  axis=(3, 4))

    x_ref[:, 0, :] = (jx.astype(f32) + oxy[:, :, 0]) / float(_WT)
    y_ref[:, 0, :] = (jy.astype(f32) + oxy[:, :, 1]) / float(_HT)
    s_ref[:, 0, :] = score


def kernel(hm, ct, rg, of):
    B, nj = hm.shape[0], hm.shape[1]
    bb = _BB
    Cm = jnp.asarray(_wresize_mat())
    spec3 = lambda c: pl.BlockSpec((bb, c, _H0, _W0), lambda b: (b, 0, 0, 0))
    x, y, s = pl.pallas_call(
        functools.partial(_decode_kernel, nj=nj, bb=bb),
        grid=(B // bb,),
        in_specs=[
            spec3(nj),
            spec3(1),
            spec3(2 * nj),
            spec3(2 * nj),
            pl.BlockSpec((_W0, _WT), lambda b: (0, 0)),
        ],
        out_specs=[
            pl.BlockSpec((bb, 1, nj), lambda b: (b, 0, 0)),
            pl.BlockSpec((bb, 1, nj), lambda b: (b, 0, 0)),
            pl.BlockSpec((bb, 1, nj), lambda b: (b, 0, 0)),
        ],
        out_shape=[jax.ShapeDtypeStruct((B, 1, nj), jnp.float32)] * 3,
        compiler_params=pltpu.CompilerParams(
            dimension_semantics=("arbitrary",),
        ),
    )(hm, ct, rg, of, Cm)
    return jnp.stack([x[:, 0], y[:, 0], s[:, 0]], axis=2).reshape(B, 3 * nj)


# speedup vs baseline: 3.3335x; 1.0075x over previous
import functools

import jax
import jax.numpy as jnp
import numpy as np
from jax.experimental import pallas as pl
from jax.experimental.pallas import tpu as pltpu

_H0, _W0 = 48, 48
_HT, _WT = 96, 96
_BB = 4


def _wresize_mat() -> np.ndarray:
    C = np.zeros((_W0, _WT), dtype=np.float64)
    for o in range(_WT):
        s = 0.5 * o - 0.25
        sc = min(max(s, 0.0), _W0 - 1.0)
        i0 = min(int(np.floor(sc)), _W0 - 2)
        w1 = sc - i0
        C[i0, o] += 1.0 - w1
        C[i0 + 1, o] += w1
    return C.astype(np.float32)


def _taps(p):
    s = 0.5 * p.astype(jnp.float32) - 0.25
    sc = jnp.clip(s, 0.0, _H0 - 1.0)
    i0 = jnp.minimum(jnp.floor(sc).astype(jnp.int32), _H0 - 2)
    w1 = sc - i0.astype(jnp.float32)
    return i0, 1.0 - w1, w1


def _w2d(py, px, shape):
    nlead = len(shape) - 2
    exp = (Ellipsis,) + (None,) * 2
    iy0, wy0, wy1 = _taps(py)
    ix0, wx0, wx1 = _taps(px)
    iy0, wy0, wy1 = iy0[exp], wy0[exp], wy1[exp]
    ix0, wx0, wx1 = ix0[exp], wx0[exp], wx1[exp]
    ri = jax.lax.broadcasted_iota(jnp.int32, shape, nlead)
    ci = jax.lax.broadcasted_iota(jnp.int32, shape, nlead + 1)
    wy = jnp.where(ri == iy0, wy0, 0.0) + jnp.where(ri == iy0 + 1, wy1, 0.0)
    wx = jnp.where(ci == ix0, wx0, 0.0) + jnp.where(ci == ix0 + 1, wx1, 0.0)
    return wy * wx


def _hresize_parity(a):
    up = jnp.concatenate([a[..., :1, :], a[..., :-1, :]], axis=-2)
    dn = jnp.concatenate([a[..., 1:, :], a[..., -1:, :]], axis=-2)
    even = 0.75 * a + 0.25 * up
    odd = 0.75 * a + 0.25 * dn
    return jnp.stack([even, odd], axis=-3)


def _decode_kernel(hm_ref, ct_ref, rg_ref, of_ref, c_ref, x_ref, y_ref, s_ref,
                   *, nj, bb):
    f32 = jnp.float32
    C = c_ref[...]
    nc = nj + 1

    hm = hm_ref[...]
    ct = ct_ref[...]
    rg = rg_ref[...]
    of = of_ref[...]

    x_in = jnp.concatenate([ct, hm], axis=1)
    a = jnp.dot(x_in.reshape(bb * nc * _H0, _W0), C,
                precision=jax.lax.Precision.HIGHEST,
                preferred_element_type=f32)
    f = _hresize_parity(a.reshape(bb, nc, _H0, _WT))
    ct_r = f[:, 0]
    hm_r = f[:, 1:]

    pshape = (2, _H0, _WT)
    yrow = (jax.lax.broadcasted_iota(jnp.int32, pshape, 1) * 2
            + jax.lax.broadcasted_iota(jnp.int32, pshape, 0))
    li = yrow * _WT + jax.lax.broadcasted_iota(jnp.int32, pshape, 2)
    big = _HT * _WT

    m = jnp.max(ct_r, axis=(1, 2, 3))
    idx = jnp.min(jnp.where(ct_r == m[:, None, None, None], li[None], big),
                  axis=(1, 2, 3))
    cy = idx // _WT
    cx = idx - cy * _WT

    wc = _w2d(cy, cx, (bb, _H0, _W0))
    rxy = jnp.sum(rg * wc[:, None], axis=(2, 3))
    rxy = rxy.reshape(bb, nj, 2)
    reg_x = jnp.clip(cx.astype(f32)[:, None] + rxy[:, :, 0] + 0.5,
                     0.0, _WT - 1.0)
    reg_y = jnp.clip(cy.astype(f32)[:, None] + rxy[:, :, 1] + 0.5,
                     0.0, _HT - 1.0)

    yrf = yrow.astype(f32)
    xrf = jax.lax.broadcasted_iota(jnp.int32, pshape, 2).astype(f32)
    d2 = ((yrf[None, None] - reg_y[:, :, None, None, None]) ** 2 + 1e-9
          + (xrf[None, None] - reg_x[:, :, None, None, None]) ** 2)
    t = hm_r * jax.lax.rsqrt(d2)
    m2 = jnp.max(t, axis=(2, 3, 4))
    idx2 = jnp.min(jnp.where(t == m2[:, :, None, None, None],
                             li[None, None], big), axis=(2, 3, 4))
    jy = idx2 // _WT
    jx = idx2 - jy * _WT
    jyf = jy.astype(f32)
    jxf = jx.astype(f32)
    d2p = (jyf - reg_y) ** 2 + 1e-9 + (jxf - reg_x) ** 2
    score = m2 * jnp.sqrt(d2p)

    wj = _w2d(jy, jx, (bb, nj, _H0, _W0))
    oxy = jnp.sum(of.reshape(bb, nj, 2, _H0, _W0) * wj[:, :, None],
                  axis=(3, 4))

    x_ref[:, 0, :] = (jx.astype(f32) + oxy[:, :, 0]) / float(_WT)
    y_ref[:, 0, :] = (jy.astype(f32) + oxy[:, :, 1]) / float(_HT)
    s_ref[:, 0, :] = score


def kernel(hm, ct, rg, of):
    B, nj = hm.shape[0], hm.shape[1]
    bb = _BB
    Cm = jnp.asarray(_wresize_mat())
    spec3 = lambda c: pl.BlockSpec((bb, c, _H0, _W0), lambda b: (b, 0, 0, 0))
    x, y, s = pl.pallas_call(
        functools.partial(_decode_kernel, nj=nj, bb=bb),
        grid=(B // bb,),
        in_specs=[
            spec3(nj),
            spec3(1),
            spec3(2 * nj),
            spec3(2 * nj),
            pl.BlockSpec((_W0, _WT), lambda b: (0, 0)),
        ],
        out_specs=[
            pl.BlockSpec((bb, 1, nj), lambda b: (b, 0, 0)),
            pl.BlockSpec((bb, 1, nj), lambda b: (b, 0, 0)),
            pl.BlockSpec((bb, 1, nj), lambda b: (b, 0, 0)),
        ],
        out_shape=[jax.ShapeDtypeStruct((B, 1, nj), jnp.float32)] * 3,
        compiler_params=pltpu.CompilerParams(
            dimension_semantics=("arbitrary",),
        ),
    )(hm, ct, rg, of, Cm)
    return jnp.stack([x[:, 0], y[:, 0], s[:, 0]], axis=2).reshape(B, 3 * nj)


# speedup vs baseline: 3.5719x; 1.0715x over previous
import functools

import jax
import jax.numpy as jnp
import numpy as np
from jax.experimental import pallas as pl
from jax.experimental.pallas import tpu as pltpu

_H0, _W0 = 48, 48
_HT, _WT = 96, 96
_BB = 4


def _wresize_mat() -> np.ndarray:
    C = np.zeros((_W0, _WT), dtype=np.float64)
    for o in range(_WT):
        s = 0.5 * o - 0.25
        sc = min(max(s, 0.0), _W0 - 1.0)
        i0 = min(int(np.floor(sc)), _W0 - 2)
        w1 = sc - i0
        C[i0, o] += 1.0 - w1
        C[i0 + 1, o] += w1
    return C.astype(np.float32)


def _src(p):
    return jnp.clip(0.5 * p.astype(jnp.float32) - 0.25, 0.0, _H0 - 1.0)


def _w2d(py, px, shape):
    nlead = len(shape) - 2
    exp = (Ellipsis,) + (None,) * 2
    sy = _src(py)[exp]
    sx = _src(px)[exp]
    f32 = jnp.float32
    ri = jax.lax.broadcasted_iota(jnp.int32, shape, nlead).astype(f32)
    ci = jax.lax.broadcasted_iota(jnp.int32, shape, nlead + 1).astype(f32)
    wy = jnp.maximum(1.0 - jnp.abs(ri - sy), 0.0)
    wx = jnp.maximum(1.0 - jnp.abs(ci - sx), 0.0)
    return wy * wx


def _hresize_parity(a):
    up = jnp.concatenate([a[..., :1, :], a[..., :-1, :]], axis=-2)
    dn = jnp.concatenate([a[..., 1:, :], a[..., -1:, :]], axis=-2)
    even = 0.75 * a + 0.25 * up
    odd = 0.75 * a + 0.25 * dn
    return jnp.stack([even, odd], axis=-3)


def _decode_kernel(hm_ref, ct_ref, rg_ref, of_ref, c_ref, x_ref, y_ref, s_ref,
                   *, nj, bb):
    f32 = jnp.float32
    C = c_ref[...]
    nc = nj + 1

    hm = hm_ref[...]
    ct = ct_ref[...]
    rg = rg_ref[...]
    of = of_ref[...]

    x_in = jnp.concatenate([ct, hm], axis=1)
    x2 = x_in.reshape(bb * nc * _H0, _W0)
    cb = C.astype(jnp.bfloat16)
    x_1 = x2.astype(jnp.bfloat16)
    r_1 = x2 - x_1.astype(f32)
    x_2 = r_1.astype(jnp.bfloat16)
    x_3 = (r_1 - x_2.astype(f32)).astype(jnp.bfloat16)
    mm = lambda u: jnp.dot(u, cb, preferred_element_type=f32)
    a = mm(x_1) + (mm(x_2) + mm(x_3))
    f = _hresize_parity(a.reshape(bb, nc, _H0, _WT))
    ct_r = f[:, 0]
    hm_r = f[:, 1:]

    pshape = (2, _H0, _WT)
    yrow = (jax.lax.broadcasted_iota(jnp.int32, pshape, 1) * 2
            + jax.lax.broadcasted_iota(jnp.int32, pshape, 0))
    li = yrow * _WT + jax.lax.broadcasted_iota(jnp.int32, pshape, 2)
    big = _HT * _WT

    m = jnp.max(ct_r, axis=(1, 2, 3))
    idx = jnp.min(jnp.where(ct_r == m[:, None, None, None], li[None], big),
                  axis=(1, 2, 3))
    cy = idx // _WT
    cx = idx - cy * _WT

    wc = _w2d(cy, cx, (bb, _H0, _W0))
    rxy = jnp.sum(rg * wc[:, None], axis=(2, 3))
    rxy = rxy.reshape(bb, nj, 2)
    reg_x = jnp.clip(cx.astype(f32)[:, None] + rxy[:, :, 0] + 0.5,
                     0.0, _WT - 1.0)
    reg_y = jnp.clip(cy.astype(f32)[:, None] + rxy[:, :, 1] + 0.5,
                     0.0, _HT - 1.0)

    yrf = yrow.astype(f32)
    xrf = jax.lax.broadcasted_iota(jnp.int32, pshape, 2).astype(f32)
    d2 = ((yrf[None, None] - reg_y[:, :, None, None, None]) ** 2 + 1e-9
          + (xrf[None, None] - reg_x[:, :, None, None, None]) ** 2)
    t = hm_r * jax.lax.rsqrt(d2)
    m2 = jnp.max(t, axis=(2, 3, 4))
    idx2 = jnp.min(jnp.where(t == m2[:, :, None, None, None],
                             li[None, None], big), axis=(2, 3, 4))
    jy = idx2 // _WT
    jx = idx2 - jy * _WT
    jyf = jy.astype(f32)
    jxf = jx.astype(f32)
    d2p = (jyf - reg_y) ** 2 + 1e-9 + (jxf - reg_x) ** 2
    score = m2 * jnp.sqrt(d2p)

    wj = _w2d(jy, jx, (bb, nj, _H0, _W0))
    oxy = jnp.sum(of.reshape(bb, nj, 2, _H0, _W0) * wj[:, :, None],
                  axis=(3, 4))

    x_ref[:, 0, :] = (jx.astype(f32) + oxy[:, :, 0]) / float(_WT)
    y_ref[:, 0, :] = (jy.astype(f32) + oxy[:, :, 1]) / float(_HT)
    s_ref[:, 0, :] = score


def kernel(hm, ct, rg, of):
    B, nj = hm.shape[0], hm.shape[1]
    bb = _BB
    Cm = jnp.asarray(_wresize_mat())
    spec3 = lambda c: pl.BlockSpec((bb, c, _H0, _W0), lambda b: (b, 0, 0, 0))
    x, y, s = pl.pallas_call(
        functools.partial(_decode_kernel, nj=nj, bb=bb),
        grid=(B // bb,),
        in_specs=[
            spec3(nj),
            spec3(1),
            spec3(2 * nj),
            spec3(2 * nj),
            pl.BlockSpec((_W0, _WT), lambda b: (0, 0)),
        ],
        out_specs=[
            pl.BlockSpec((bb, 1, nj), lambda b: (b, 0, 0)),
            pl.BlockSpec((bb, 1, nj), lambda b: (b, 0, 0)),
            pl.BlockSpec((bb, 1, nj), lambda b: (b, 0, 0)),
        ],
        out_shape=[jax.ShapeDtypeStruct((B, 1, nj), jnp.float32)] * 3,
        compiler_params=pltpu.CompilerParams(
            dimension_semantics=("arbitrary",),
        ),
    )(hm, ct, rg, of, Cm)
    return jnp.stack([x[:, 0], y[:, 0], s[:, 0]], axis=2).reshape(B, 3 * nj)
